# baseline (speedup 1.0000x reference)
import functools

import jax
import jax.numpy as jnp
from jax import lax
from jax.experimental import pallas as pl
from jax.experimental.pallas import tpu as pltpu
from jax.experimental.pallas import tpu_sc as plsc

D_MODEL = 1024
NUM_WAVES = 3
N_TOKENS = 4 * 8192
NUM_CORES = 2
NUM_SUBCORES = 16
NUM_WORKERS = NUM_CORES * NUM_SUBCORES
TOK_PER_W = N_TOKENS // NUM_WORKERS
LANES = 16
NGROUP = TOK_PER_W // LANES
NBLK = 4
BLK_GROUPS = NGROUP // NBLK
BLK_ROWS = BLK_GROUPS + 1
BLK_STRIDE = BLK_GROUPS * LANES + LANES
REP = 48
NREPLICA = 8

_mesh = plsc.VectorSubcoreMesh(core_axis_name="c", subcore_axis_name="s")


@functools.partial(
    pl.kernel,
    mesh=_mesh,
    out_type=jax.ShapeDtypeStruct((N_TOKENS, D_MODEL), jnp.float32),
    scratch_types=[
        pltpu.VMEM((TOK_PER_W,), jnp.float32),
        pltpu.VMEM((TOK_PER_W,), jnp.float32),
        pltpu.VMEM((TOK_PER_W,), jnp.float32),
        pltpu.VMEM((NBLK * NUM_WAVES * BLK_STRIDE + LANES,), jnp.int32),
        pltpu.VMEM((NBLK * NUM_WAVES * BLK_ROWS, LANES), jnp.int32),
        pltpu.VMEM((REP, D_MODEL), jnp.float32),
        pltpu.SemaphoreType.DMA,
        pltpu.SemaphoreType.DMA,
    ],
)
def _sc_kernel(m0_h, m1_h, m2_h, rep_h, out_h,
               m0_v, m1_v, m2_v, flat_v, list2_v, rep_v, sem, rsem):
    wid = lax.axis_index("s") * NUM_CORES + lax.axis_index("c")
    base = wid * TOK_PER_W

    pltpu.async_copy(
        rep_h.at[pl.ds((wid % NREPLICA) * REP, REP)], rep_v, rsem)
    pltpu.sync_copy(m0_h.at[pl.ds(base, TOK_PER_W)], m0_v)
    pltpu.sync_copy(m1_h.at[pl.ds(base, TOK_PER_W)], m1_v)
    pltpu.sync_copy(m2_h.at[pl.ds(base, TOK_PER_W)], m2_v)

    one = jnp.full((LANES,), 1, jnp.int32)
    zero = jnp.full((LANES,), 0, jnp.int32)
    two = jnp.full((LANES,), 2, jnp.int32)

    def make_compact_step(blk):
        def compact_step(i, counts):
            n0, n1, n2 = counts
            g = blk * BLK_GROUPS + i
            a0 = m0_v[pl.ds(g * LANES, LANES)]
            a1 = m1_v[pl.ds(g * LANES, LANES)]
            a2 = m2_v[pl.ds(g * LANES, LANES)]
            lbl = jnp.where(a1 > a0, one, zero)
            mx = jnp.maximum(a0, a1)
            lbl = jnp.where(a2 > mx, two, lbl)
            gbase = base + g * LANES
            for l in range(LANES):
                lv = lbl[l]
                is0 = lv == 0
                is1 = lv == 1
                nsel = jnp.where(is0, n0, jnp.where(is1, n1, n2))
                off = (blk * NUM_WAVES + lv) * BLK_STRIDE + nsel
                flat_v[pl.ds(off, LANES)] = zero + (gbase + l)
                n0 = n0 + jnp.where(is0, 1, 0)
                n1 = n1 + jnp.where(is1, 1, 0)
                n2 = n2 + jnp.where(jnp.logical_or(is0, is1), 0, 1)
            return (n0, n1, n2)

        return compact_step

    total_rows = jnp.int32(0)
    for blk in range(NBLK):
        counts = lax.fori_loop(
            0, BLK_GROUPS, make_compact_step(blk),
            (jnp.int32(0), jnp.int32(0), jnp.int32(0)))

        if blk == 0:
            pltpu.make_async_copy(
                rep_h.at[pl.ds((wid % NREPLICA) * REP, REP)], rep_v, rsem
            ).wait()

        for c in range(NUM_WAVES):
            n_c = counts[c]
            r = blk * NUM_WAVES + c

            @pl.when(n_c > 0)
            def _(c=c, n_c=n_c, r=r):
                fbase = r * BLK_STRIDE
                lastv = flat_v[pl.ds(fbase + n_c - 1, LANES)]
                flat_v[pl.ds(fbase + n_c, LANES)] = zero + lastv[0]
                nrows = (n_c + LANES - 1) // LANES

                def row_step(j, carry):
                    list2_v[r * BLK_ROWS + j, :] = (
                        flat_v[pl.ds(fbase + j * LANES, LANES)])
                    return carry

                lax.fori_loop(0, nrows, row_step, 0)

                src = rep_v.at[pl.ds(c * LANES, LANES)]

                def fire_step(j, carry):
                    pltpu.async_copy(
                        src, out_h.at[list2_v.at[r * BLK_ROWS + j]], sem)
                    return carry

                lax.fori_loop(0, nrows, fire_step, 0)

            total_rows = total_rows + (n_c + LANES - 1) // LANES


    def drain_step(j, carry):
        pltpu.make_async_copy(
            rep_v.at[pl.ds(0, LANES)], out_h.at[list2_v.at[0]], sem).wait()
        return carry

    lax.fori_loop(0, total_rows, drain_step, 0)


def kernel(wave_mask, wave_embedding):
    B, S, W = wave_mask.shape
    maskT = wave_mask.reshape(B * S, W).T
    rep48 = jnp.repeat(wave_embedding, LANES, axis=0)
    rep_all = jnp.tile(rep48, (NREPLICA, 1))
    out = _sc_kernel(maskT[0], maskT[1], maskT[2], rep_all)
    return out.reshape(B, S, D_MODEL)


# speedup vs baseline: 1.0509x; 1.0509x over previous
import functools

import jax
import jax.numpy as jnp
from jax import lax
from jax.experimental import pallas as pl
from jax.experimental.pallas import tpu as pltpu
from jax.experimental.pallas import tpu_sc as plsc

D_MODEL = 1024
NUM_WAVES = 3
N_TOKENS = 4 * 8192
NUM_CORES = 2
NUM_SUBCORES = 16
NUM_WORKERS = NUM_CORES * NUM_SUBCORES
TOK_PER_W = N_TOKENS // NUM_WORKERS
LANES = 16
NGROUP = TOK_PER_W // LANES
LIST_ROWS = NGROUP + 1
FLAT_STRIDE = TOK_PER_W + LANES
REP = 48
NREPLICA = 8

_mesh = plsc.VectorSubcoreMesh(core_axis_name="c", subcore_axis_name="s")


@functools.partial(
    pl.kernel,
    mesh=_mesh,
    out_type=jax.ShapeDtypeStruct((N_TOKENS, D_MODEL), jnp.float32),
    scratch_types=[
        pltpu.VMEM((TOK_PER_W,), jnp.float32),
        pltpu.VMEM((TOK_PER_W,), jnp.float32),
        pltpu.VMEM((TOK_PER_W,), jnp.float32),
        pltpu.VMEM((NUM_WAVES * FLAT_STRIDE + LANES,), jnp.int32),
        pltpu.VMEM((NUM_WAVES * LIST_ROWS, LANES), jnp.int32),
        pltpu.VMEM((REP, D_MODEL), jnp.float32),
        pltpu.SemaphoreType.DMA,
        pltpu.SemaphoreType.DMA,
    ],
)
def _sc_kernel(m0_h, m1_h, m2_h, rep_h, out_h,
               m0_v, m1_v, m2_v, flat_v, list2_v, rep_v, sem, rsem):
    wid = lax.axis_index("s") * NUM_CORES + lax.axis_index("c")
    base = wid * TOK_PER_W

    pltpu.async_copy(
        rep_h.at[pl.ds((wid % NREPLICA) * REP, REP)], rep_v, rsem)
    pltpu.sync_copy(m0_h.at[pl.ds(base, TOK_PER_W)], m0_v)
    pltpu.sync_copy(m1_h.at[pl.ds(base, TOK_PER_W)], m1_v)
    pltpu.sync_copy(m2_h.at[pl.ds(base, TOK_PER_W)], m2_v)

    one = jnp.full((LANES,), 1, jnp.int32)
    zero = jnp.full((LANES,), 0, jnp.int32)
    two = jnp.full((LANES,), 2, jnp.int32)

    def compact_step(i, counts):
        n0, n1, n2 = counts
        a0 = m0_v[pl.ds(i * LANES, LANES)]
        a1 = m1_v[pl.ds(i * LANES, LANES)]
        a2 = m2_v[pl.ds(i * LANES, LANES)]
        lbl = jnp.where(a1 > a0, one, zero)
        mx = jnp.maximum(a0, a1)
        lbl = jnp.where(a2 > mx, two, lbl)
        gbase = base + i * LANES
        for l in range(LANES):
            lv = lbl[l]
            is0 = lv == 0
            is1 = lv == 1
            nsel = jnp.where(is0, n0, jnp.where(is1, n1, n2))
            off = lv * FLAT_STRIDE + nsel
            flat_v[pl.ds(off, LANES)] = zero + (gbase + l)
            n0 = n0 + jnp.where(is0, 1, 0)
            n1 = n1 + jnp.where(is1, 1, 0)
            n2 = n2 + jnp.where(jnp.logical_or(is0, is1), 0, 1)
        return (n0, n1, n2)

    counts = lax.fori_loop(
        0, NGROUP, compact_step,
        (jnp.int32(0), jnp.int32(0), jnp.int32(0)))

    pltpu.make_async_copy(
        rep_h.at[pl.ds((wid % NREPLICA) * REP, REP)], rep_v, rsem).wait()

    for c in range(NUM_WAVES):
        n_c = counts[c]

        @pl.when(n_c > 0)
        def _(c=c, n_c=n_c):
            lastv = flat_v[pl.ds(c * FLAT_STRIDE + n_c - 1, LANES)]
            flat_v[pl.ds(c * FLAT_STRIDE + n_c, LANES)] = zero + lastv[0]
            nrows = (n_c + LANES - 1) // LANES

            def row_step(j, carry):
                list2_v[c * LIST_ROWS + j, :] = (
                    flat_v[pl.ds(c * FLAT_STRIDE + j * LANES, LANES)])
                return carry

            lax.fori_loop(0, nrows, row_step, 0)

            src = rep_v.at[pl.ds(c * LANES, LANES)]

            def fire_step(j, carry):
                pltpu.async_copy(
                    src, out_h.at[list2_v.at[c * LIST_ROWS + j]], sem)
                return carry

            lax.fori_loop(0, nrows, fire_step, 0)

    total_rows = sum(
        (counts[c] + LANES - 1) // LANES for c in range(NUM_WAVES))

    def drain_step(j, carry):
        pltpu.make_async_copy(
            rep_v.at[pl.ds(0, LANES)], out_h.at[list2_v.at[0]], sem).wait()
        return carry

    lax.fori_loop(0, total_rows, drain_step, 0)


def kernel(wave_mask, wave_embedding):
    B, S, W = wave_mask.shape
    maskT = wave_mask.reshape(B * S, W).T
    rep48 = jnp.repeat(wave_embedding, LANES, axis=0)
    rep_all = jnp.tile(rep48, (NREPLICA, 1))
    out = _sc_kernel(maskT[0], maskT[1], maskT[2], rep_all)
    return out.reshape(B, S, D_MODEL)


# speedup vs baseline: 1.0853x; 1.0328x over previous
import functools

import jax
import jax.numpy as jnp
from jax import lax
from jax.experimental import pallas as pl
from jax.experimental.pallas import tpu as pltpu
from jax.experimental.pallas import tpu_sc as plsc

D_MODEL = 1024
NUM_WAVES = 3
N_TOKENS = 4 * 8192
NUM_CORES = 2
NUM_SUBCORES = 16
NUM_WORKERS = NUM_CORES * NUM_SUBCORES
TOK_PER_W = N_TOKENS // NUM_WORKERS
LANES = 16
NGROUP = TOK_PER_W // LANES
LIST_ROWS = NGROUP + 1
FLAT_STRIDE = TOK_PER_W + LANES
REP = 48
NREPLICA = 8

_mesh = plsc.VectorSubcoreMesh(core_axis_name="c", subcore_axis_name="s")


@functools.partial(
    pl.kernel,
    mesh=_mesh,
    out_type=jax.ShapeDtypeStruct((N_TOKENS, D_MODEL), jnp.float32),
    scratch_types=[
        pltpu.VMEM((TOK_PER_W,), jnp.float32),
        pltpu.VMEM((TOK_PER_W,), jnp.float32),
        pltpu.VMEM((TOK_PER_W,), jnp.float32),
        pltpu.VMEM((NUM_WAVES * FLAT_STRIDE + LANES,), jnp.int32),
        pltpu.VMEM((NUM_WAVES * LIST_ROWS, LANES), jnp.int32),
        pltpu.VMEM((REP, D_MODEL), jnp.float32),
        pltpu.SemaphoreType.DMA,
        pltpu.SemaphoreType.DMA,
        pltpu.SemaphoreType.DMA,
    ],
)
def _sc_kernel(m0_h, m1_h, m2_h, rep_h, out_h,
               m0_v, m1_v, m2_v, flat_v, list2_v, rep_v, sem, rsem, msem):
    wid = lax.axis_index("s") * NUM_CORES + lax.axis_index("c")
    base = wid * TOK_PER_W

    pltpu.async_copy(
        rep_h.at[pl.ds((wid % NREPLICA) * REP, REP)], rep_v, rsem)
    pltpu.async_copy(m0_h.at[pl.ds(base, TOK_PER_W)], m0_v, msem)
    pltpu.async_copy(m1_h.at[pl.ds(base, TOK_PER_W)], m1_v, msem)
    pltpu.async_copy(m2_h.at[pl.ds(base, TOK_PER_W)], m2_v, msem)
    for _ in range(3):
        pltpu.make_async_copy(
            m0_h.at[pl.ds(base, TOK_PER_W)], m0_v, msem).wait()

    one = jnp.full((LANES,), 1, jnp.int32)
    zero = jnp.full((LANES,), 0, jnp.int32)
    two = jnp.full((LANES,), 2, jnp.int32)

    def compact_step(i, counts):
        n0, n1, n2 = counts
        a0 = m0_v[pl.ds(i * LANES, LANES)]
        a1 = m1_v[pl.ds(i * LANES, LANES)]
        a2 = m2_v[pl.ds(i * LANES, LANES)]
        lbl = jnp.where(a1 > a0, one, zero)
        mx = jnp.maximum(a0, a1)
        lbl = jnp.where(a2 > mx, two, lbl)
        gbase = base + i * LANES
        for l in range(LANES):
            lv = lbl[l]
            is0 = lv == 0
            is1 = lv == 1
            nsel = jnp.where(is0, n0, jnp.where(is1, n1, n2))
            off = lv * FLAT_STRIDE + nsel
            flat_v[pl.ds(off, LANES)] = zero + (gbase + l)
            n0 = n0 + jnp.where(is0, 1, 0)
            n1 = n1 + jnp.where(is1, 1, 0)
            n2 = n2 + jnp.where(jnp.logical_or(is0, is1), 0, 1)
        return (n0, n1, n2)

    counts = lax.fori_loop(
        0, NGROUP, compact_step,
        (jnp.int32(0), jnp.int32(0), jnp.int32(0)))

    pltpu.make_async_copy(
        rep_h.at[pl.ds((wid % NREPLICA) * REP, REP)], rep_v, rsem).wait()

    for c in range(NUM_WAVES):
        n_c = counts[c]

        @pl.when(n_c > 0)
        def _(c=c, n_c=n_c):
            lastv = flat_v[pl.ds(c * FLAT_STRIDE + n_c - 1, LANES)]
            flat_v[pl.ds(c * FLAT_STRIDE + n_c, LANES)] = zero + lastv[0]
            nrows = (n_c + LANES - 1) // LANES

            def row_step(j, carry):
                list2_v[c * LIST_ROWS + j, :] = (
                    flat_v[pl.ds(c * FLAT_STRIDE + j * LANES, LANES)])
                return carry

            lax.fori_loop(0, nrows, row_step, 0)

            src = rep_v.at[pl.ds(c * LANES, LANES)]

            def fire_step(j, carry):
                pltpu.async_copy(
                    src, out_h.at[list2_v.at[c * LIST_ROWS + j]], sem)
                return carry

            lax.fori_loop(0, nrows, fire_step, 0)

    total_rows = sum(
        (counts[c] + LANES - 1) // LANES for c in range(NUM_WAVES))

    def drain_step(j, carry):
        pltpu.make_async_copy(
            rep_v.at[pl.ds(0, LANES)], out_h.at[list2_v.at[0]], sem).wait()
        return carry

    lax.fori_loop(0, total_rows, drain_step, 0)


def kernel(wave_mask, wave_embedding):
    B, S, W = wave_mask.shape
    maskT = wave_mask.reshape(B * S, W).T
    rep48 = jnp.repeat(wave_embedding, LANES, axis=0)
    rep_all = jnp.tile(rep48, (NREPLICA, 1))
    out = _sc_kernel(maskT[0], maskT[1], maskT[2], rep_all)
    return out.reshape(B, S, D_MODEL)


# speedup vs baseline: 1.1118x; 1.0244x over previous
import functools

import jax
import jax.numpy as jnp
from jax import lax
from jax.experimental import pallas as pl
from jax.experimental.pallas import tpu as pltpu
from jax.experimental.pallas import tpu_sc as plsc

D_MODEL = 1024
NUM_WAVES = 3
N_TOKENS = 4 * 8192
NUM_CORES = 2
NUM_SUBCORES = 16
NUM_WORKERS = NUM_CORES * NUM_SUBCORES
TOK_PER_W = N_TOKENS // NUM_WORKERS
LANES = 16
NGROUP = TOK_PER_W // LANES
LIST_ROWS = NGROUP + 1
FLAT_STRIDE = TOK_PER_W + LANES
REP = 48
NREPLICA = 8

_mesh = plsc.VectorSubcoreMesh(core_axis_name="c", subcore_axis_name="s")


@functools.partial(
    pl.kernel,
    mesh=_mesh,
    out_type=jax.ShapeDtypeStruct((N_TOKENS, D_MODEL), jnp.float32),
    scratch_types=[
        pltpu.VMEM((TOK_PER_W,), jnp.float32),
        pltpu.VMEM((TOK_PER_W,), jnp.float32),
        pltpu.VMEM((TOK_PER_W,), jnp.float32),
        pltpu.VMEM((NUM_WAVES * FLAT_STRIDE + LANES,), jnp.int32),
        pltpu.VMEM((NUM_WAVES * LIST_ROWS, LANES), jnp.int32),
        pltpu.VMEM((REP, D_MODEL), jnp.float32),
        pltpu.SemaphoreType.DMA,
        pltpu.SemaphoreType.DMA,
        pltpu.SemaphoreType.DMA,
    ],
)
def _sc_kernel(m0_h, m1_h, m2_h, rep_h, out_h,
               m0_v, m1_v, m2_v, flat_v, list2_v, rep_v, sem, rsem, msem):
    wid = lax.axis_index("s") * NUM_CORES + lax.axis_index("c")
    base = wid * TOK_PER_W

    pltpu.async_copy(
        rep_h.at[pl.ds((wid % NREPLICA) * REP, REP)], rep_v, rsem)
    pltpu.async_copy(m0_h.at[pl.ds(base, TOK_PER_W)], m0_v, msem)
    pltpu.async_copy(m1_h.at[pl.ds(base, TOK_PER_W)], m1_v, msem)
    pltpu.async_copy(m2_h.at[pl.ds(base, TOK_PER_W)], m2_v, msem)
    for _ in range(3):
        pltpu.make_async_copy(
            m0_h.at[pl.ds(base, TOK_PER_W)], m0_v, msem).wait()

    one = jnp.full((LANES,), 1, jnp.int32)
    zero = jnp.full((LANES,), 0, jnp.int32)
    two = jnp.full((LANES,), 2, jnp.int32)

    def compact_step(i, counts):
        n0, n1, n2 = counts
        a0 = m0_v[pl.ds(i * LANES, LANES)]
        a1 = m1_v[pl.ds(i * LANES, LANES)]
        a2 = m2_v[pl.ds(i * LANES, LANES)]
        lbl = jnp.where(a1 > a0, one, zero)
        mx = jnp.maximum(a0, a1)
        lbl = jnp.where(a2 > mx, two, lbl)
        gbase = base + i * LANES
        for l in range(LANES):
            lv = lbl[l]
            is0 = lv == 0
            is1 = lv == 1
            nsel = jnp.where(is0, n0, jnp.where(is1, n1, n2))
            off = lv * FLAT_STRIDE + nsel
            flat_v[pl.ds(off, LANES)] = zero + (gbase + l)
            n0 = n0 + jnp.where(is0, 1, 0)
            n1 = n1 + jnp.where(is1, 1, 0)
            n2 = n2 + jnp.where(jnp.logical_or(is0, is1), 0, 1)
        return (n0, n1, n2)

    def relayout_and_fire(c, row_lo, row_hi):
        src = rep_v.at[pl.ds(c * LANES, LANES)]

        def row_step(j, carry):
            list2_v[c * LIST_ROWS + j, :] = (
                flat_v[pl.ds(c * FLAT_STRIDE + j * LANES, LANES)])
            return carry

        lax.fori_loop(row_lo, row_hi, row_step, 0)

        def fire_step(j, carry):
            pltpu.async_copy(
                src, out_h.at[list2_v.at[c * LIST_ROWS + j]], sem)
            return carry

        lax.fori_loop(row_lo, row_hi, fire_step, 0)

    counts_half = lax.fori_loop(
        0, NGROUP // 2, compact_step,
        (jnp.int32(0), jnp.int32(0), jnp.int32(0)))

    pltpu.make_async_copy(
        rep_h.at[pl.ds((wid % NREPLICA) * REP, REP)], rep_v, rsem).wait()

    for c in range(NUM_WAVES):
        nfull = counts_half[c] // LANES

        @pl.when(nfull > 0)
        def _(c=c, nfull=nfull):
            relayout_and_fire(c, 0, nfull)

    counts = lax.fori_loop(
        NGROUP // 2, NGROUP, compact_step, counts_half)

    for c in range(NUM_WAVES):
        n_c = counts[c]

        @pl.when(n_c > 0)
        def _(c=c, n_c=n_c, start_row=counts_half[c] // LANES):
            lastv = flat_v[pl.ds(c * FLAT_STRIDE + n_c - 1, LANES)]
            flat_v[pl.ds(c * FLAT_STRIDE + n_c, LANES)] = zero + lastv[0]
            nrows = (n_c + LANES - 1) // LANES
            relayout_and_fire(c, start_row, nrows)

    total_rows = sum(
        (counts[c] + LANES - 1) // LANES for c in range(NUM_WAVES))

    def drain_step(j, carry):
        pltpu.make_async_copy(
            rep_v.at[pl.ds(0, LANES)], out_h.at[list2_v.at[0]], sem).wait()
        return carry

    lax.fori_loop(0, total_rows, drain_step, 0)


def kernel(wave_mask, wave_embedding):
    B, S, W = wave_mask.shape
    maskT = wave_mask.reshape(B * S, W).T
    rep48 = jnp.repeat(wave_embedding, LANES, axis=0)
    rep_all = jnp.tile(rep48, (NREPLICA, 1))
    out = _sc_kernel(maskT[0], maskT[1], maskT[2], rep_all)
    return out.reshape(B, S, D_MODEL)
